# baseline (speedup 1.0000x reference)
import functools

import jax
import jax.numpy as jnp
from jax import lax
from jax.experimental import pallas as pl
from jax.experimental.pallas import tpu as pltpu
from jax.experimental.pallas import tpu_sc as plsc

N = 10000
E = 320000
D = 128

NC = 2
NS = 16
NW = NC * NS
CHUNK = 128
CHUNKS = 80
EP = NW * CHUNKS * CHUNK
ACC_ROWS = 10240
ROWS_PER_SUB = ACC_ROWS // NS
OUT_PER_SUB = 624
OUT_LAST = N - 15 * OUT_PER_SUB
EPS = 1e-5
NBUF = 2
UNROLL = 8
HALF = CHUNKS // 2


@functools.lru_cache(maxsize=None)
def _sc_kernels():
    mesh = plsc.VectorSubcoreMesh(core_axis_name="c", subcore_axis_name="s",
                                  num_cores=NC, num_subcores=NS)

    @functools.partial(
        pl.kernel,
        out_type=jax.ShapeDtypeStruct((NC, N, D), jnp.float32),
        mesh=mesh,
        scratch_types=[
            pltpu.VMEM((HALF, CHUNK), jnp.int32),
            pltpu.VMEM((HALF, CHUNK), jnp.int32),
            pltpu.VMEM((NBUF, CHUNK, D), jnp.float32),
            pltpu.VMEM_SHARED((ACC_ROWS, D), jnp.float32),
            pltpu.SemaphoreType.DMA,
            pltpu.SemaphoreType.DMA,
        ],
    )
    def sc_agg(table_hbm, src_hbm, dst_hbm, zeros_hbm, out_hbm,
               srcidx, dstidx, rows, acc, sem0, sem1):
        sems = (sem0, sem1)
        c = lax.axis_index("c")
        s = lax.axis_index("s")
        wid = c * NS + s
        pltpu.sync_copy(zeros_hbm.at[pl.ds(s * ROWS_PER_SUB, ROWS_PER_SUB)],
                        acc.at[pl.ds(s * ROWS_PER_SUB, ROWS_PER_SUB)])
        plsc.subcore_barrier()

        for p in range(2):
            pltpu.sync_copy(src_hbm.at[wid, pl.ds(p * HALF, HALF)], srcidx)
            pltpu.sync_copy(dst_hbm.at[wid, pl.ds(p * HALF, HALF)], dstidx)
            for b in range(NBUF):
                pltpu.async_copy(table_hbm.at[srcidx.at[b]], rows.at[b], sems[b])

            def outer(o, carry):
                for i in range(UNROLL):
                    j = o * UNROLL + i
                    b = i % NBUF
                    pltpu.make_async_copy(table_hbm.at[srcidx.at[j]],
                                          rows.at[b], sems[b]).wait()
                    pltpu.sync_copy(rows.at[b], acc.at[dstidx.at[j]], add=True)

                    @pl.when(j < HALF - NBUF)
                    def _():
                        pltpu.async_copy(table_hbm.at[srcidx.at[j + NBUF]],
                                         rows.at[b], sems[b])
                return carry

            lax.fori_loop(0, HALF // UNROLL, outer, 0)
        plsc.subcore_barrier()

        @pl.when(s < NS - 1)
        def _():
            pltpu.sync_copy(acc.at[pl.ds(s * OUT_PER_SUB, OUT_PER_SUB)],
                            out_hbm.at[c, pl.ds(s * OUT_PER_SUB, OUT_PER_SUB)])

        @pl.when(s == NS - 1)
        def _():
            pltpu.sync_copy(acc.at[pl.ds(15 * OUT_PER_SUB, OUT_LAST)],
                            out_hbm.at[c, pl.ds(15 * OUT_PER_SUB, OUT_LAST)])

    @functools.partial(
        pl.kernel,
        out_type=jax.ShapeDtypeStruct((NC, N, D), jnp.float32),
        mesh=mesh,
        scratch_types=[
            pltpu.VMEM((CHUNKS, CHUNK), jnp.int32),
            pltpu.VMEM((CHUNK, D), jnp.float32),
            pltpu.VMEM_SHARED((ACC_ROWS, D), jnp.float32),
            pltpu.SemaphoreType.DMA,
        ],
    )
    def sc_deg(dst_hbm, ones_hbm, zeros_hbm, out_hbm, dstidx, ones_v, dacc, sem):
        c = lax.axis_index("c")
        s = lax.axis_index("s")
        wid = c * NS + s
        pltpu.sync_copy(zeros_hbm.at[pl.ds(s * ROWS_PER_SUB, ROWS_PER_SUB)],
                        dacc.at[pl.ds(s * ROWS_PER_SUB, ROWS_PER_SUB)])
        pltpu.sync_copy(ones_hbm, ones_v)
        pltpu.sync_copy(dst_hbm.at[wid], dstidx)
        plsc.subcore_barrier()

        def fire(j, carry):
            pltpu.async_copy(ones_v, dacc.at[dstidx.at[j]], sem, add=True)
            return carry

        lax.fori_loop(0, CHUNKS, fire, 0)

        def drain(j, carry):
            pltpu.make_async_copy(ones_v, dacc.at[dstidx.at[j]], sem).wait()
            return carry

        lax.fori_loop(0, CHUNKS, drain, 0)
        plsc.subcore_barrier()

        @pl.when(s < NS - 1)
        def _():
            pltpu.sync_copy(dacc.at[pl.ds(s * OUT_PER_SUB, OUT_PER_SUB)],
                            out_hbm.at[c, pl.ds(s * OUT_PER_SUB, OUT_PER_SUB)])

        @pl.when(s == NS - 1)
        def _():
            pltpu.sync_copy(dacc.at[pl.ds(15 * OUT_PER_SUB, OUT_LAST)],
                            out_hbm.at[c, pl.ds(15 * OUT_PER_SUB, OUT_LAST)])

    return sc_agg, sc_deg


def _sc_agg(*args):
    return _sc_kernels()[0](*args)


def _sc_deg(*args):
    return _sc_kernels()[1](*args)


def _dinv_from(dp0, dp1):
    deg = dp0[:, 0:1] + dp1[:, 0:1] + 1.0
    return lax.rsqrt(deg)


def _bn(x, w, b):
    mean = jnp.mean(x, axis=0, keepdims=True)
    var = jnp.mean((x - mean) ** 2, axis=0, keepdims=True)
    return (x - mean) * lax.rsqrt(var + EPS) * w + b


def _tc_feat_body(x_ref, bfw_ref, bfb_ref, Wf_ref, bf_ref,
                  bw_ref, bb_ref, W_ref, b_ref, dp0_ref, dp1_ref, t0_ref):
    dinv = _dinv_from(dp0_ref[...], dp1_ref[...])
    h = _bn(x_ref[...], bfw_ref[...], bfb_ref[...])
    h = jnp.maximum(jnp.dot(h, Wf_ref[...],
                            preferred_element_type=jnp.float32) + bf_ref[...], 0.0)
    hb = _bn(h, bw_ref[...], bb_ref[...])
    t0_ref[...] = dinv * (jnp.dot(hb, W_ref[...],
                                  preferred_element_type=jnp.float32) + b_ref[...])


def _tc_mid_body(a0_ref, a1_ref, t_ref, dp0_ref, dp1_ref,
                 bw_ref, bb_ref, W_ref, b_ref, out_ref):
    dinv = _dinv_from(dp0_ref[...], dp1_ref[...])
    h = jnp.maximum(dinv * (a0_ref[...] + a1_ref[...] + t_ref[...]), 0.0)
    hb = _bn(h, bw_ref[...], bb_ref[...])
    out_ref[...] = dinv * (jnp.dot(hb, W_ref[...],
                                   preferred_element_type=jnp.float32) + b_ref[...])


def _tc_final_body(a0_ref, a1_ref, t_ref, dp0_ref, dp1_ref, out_ref):
    dinv = _dinv_from(dp0_ref[...], dp1_ref[...])
    out_ref[...] = jnp.maximum(dinv * (a0_ref[...] + a1_ref[...] + t_ref[...]), 0.0)


def _tc(body, *args):
    return pl.pallas_call(
        body, out_shape=jax.ShapeDtypeStruct((N, D), jnp.float32))(*args)


def kernel(x, edge_index, bn_feat_w, bn_feat_b, W_feat, b_feat,
           bn_ws, bn_bs, Ws, bs):
    f32 = jnp.float32
    pad = EP - E
    src3 = jnp.concatenate(
        [edge_index[0], jnp.zeros((pad,), jnp.int32)]).reshape(NW, CHUNKS, CHUNK)
    dst3 = jnp.concatenate(
        [edge_index[1], jnp.full((pad,), N, jnp.int32)]).reshape(NW, CHUNKS, CHUNK)
    zeros_d = jnp.zeros((ACC_ROWS, D), f32)
    ones_d = jnp.ones((CHUNK, D), f32)

    degp = _sc_deg(dst3, ones_d, zeros_d)
    dp0, dp1 = degp[0], degp[1]

    row = lambda v: v.reshape(1, D)
    t = _tc(_tc_feat_body, x, row(bn_feat_w), row(bn_feat_b), W_feat,
            row(b_feat), row(bn_ws[0]), row(bn_bs[0]), Ws[0], row(bs[0]),
            dp0, dp1)
    for i in range(1, 3):
        a = _sc_agg(t, src3, dst3, zeros_d)
        t = _tc(_tc_mid_body, a[0], a[1], t, dp0, dp1,
                row(bn_ws[i]), row(bn_bs[i]), Ws[i], row(bs[i]))
    a = _sc_agg(t, src3, dst3, zeros_d)
    return _tc(_tc_final_body, a[0], a[1], t, dp0, dp1)


# speedup vs baseline: 1.9725x; 1.9725x over previous
import functools

import jax
import jax.numpy as jnp
from jax import lax
from jax.experimental import pallas as pl
from jax.experimental.pallas import tpu as pltpu
from jax.experimental.pallas import tpu_sc as plsc

N = 10000
E = 320000
D = 128

NC = 2
NS = 16
NW = NC * NS
CHUNK = 128
CHUNKS = 80
EP = NW * CHUNKS * CHUNK
ACC_ROWS = 10240
ROWS_PER_SUB = ACC_ROWS // NS
OUT_PER_SUB = 624
OUT_LAST = N - 15 * OUT_PER_SUB
EPS = 1e-5
REP = 4
NBUF = 2
UNROLL = 8
HALF = CHUNKS // 2


@functools.lru_cache(maxsize=None)
def _sc_kernels():
    mesh = plsc.VectorSubcoreMesh(core_axis_name="c", subcore_axis_name="s",
                                  num_cores=NC, num_subcores=NS)

    @functools.partial(
        pl.kernel,
        out_type=jax.ShapeDtypeStruct((NC, N, D), jnp.float32),
        mesh=mesh,
        scratch_types=[
            pltpu.VMEM((HALF, CHUNK), jnp.int32),
            pltpu.VMEM((HALF, CHUNK), jnp.int32),
            pltpu.VMEM((NBUF, CHUNK, D), jnp.float32),
            pltpu.VMEM_SHARED((ACC_ROWS, D), jnp.float32),
            pltpu.SemaphoreType.DMA,
            pltpu.SemaphoreType.DMA,
        ],
    )
    def sc_agg(table_hbm, src_hbm, dst_hbm, zeros_hbm, out_hbm,
               srcidx, dstidx, rows, acc, sem0, sem1):
        sems = (sem0, sem1)
        c = lax.axis_index("c")
        s = lax.axis_index("s")
        wid = c * NS + s
        pltpu.sync_copy(zeros_hbm.at[pl.ds(s * ROWS_PER_SUB, ROWS_PER_SUB)],
                        acc.at[pl.ds(s * ROWS_PER_SUB, ROWS_PER_SUB)])
        plsc.subcore_barrier()

        for p in range(2):
            pltpu.sync_copy(src_hbm.at[wid, pl.ds(p * HALF, HALF)], srcidx)
            pltpu.sync_copy(dst_hbm.at[wid, pl.ds(p * HALF, HALF)], dstidx)
            for b in range(NBUF):
                pltpu.async_copy(table_hbm.at[srcidx.at[b]], rows.at[b], sems[b])

            def outer(o, carry):
                for i in range(UNROLL):
                    j = o * UNROLL + i
                    b = i % NBUF
                    pltpu.make_async_copy(table_hbm.at[srcidx.at[j]],
                                          rows.at[b], sems[b]).wait()
                    pltpu.sync_copy(rows.at[b], acc.at[dstidx.at[j]], add=True)

                    @pl.when(j < HALF - NBUF)
                    def _():
                        pltpu.async_copy(table_hbm.at[srcidx.at[j + NBUF]],
                                         rows.at[b], sems[b])
                return carry

            lax.fori_loop(0, HALF // UNROLL, outer, 0)
        plsc.subcore_barrier()

        @pl.when(s < NS - 1)
        def _():
            pltpu.sync_copy(acc.at[pl.ds(s * OUT_PER_SUB, OUT_PER_SUB)],
                            out_hbm.at[c, pl.ds(s * OUT_PER_SUB, OUT_PER_SUB)])

        @pl.when(s == NS - 1)
        def _():
            pltpu.sync_copy(acc.at[pl.ds(15 * OUT_PER_SUB, OUT_LAST)],
                            out_hbm.at[c, pl.ds(15 * OUT_PER_SUB, OUT_LAST)])

    @functools.partial(
        pl.kernel,
        out_type=jax.ShapeDtypeStruct((NC, N, D), jnp.float32),
        mesh=mesh,
        scratch_types=[
            pltpu.VMEM((CHUNKS, CHUNK), jnp.int32),
            pltpu.VMEM((CHUNK, D), jnp.float32),
            pltpu.VMEM_SHARED((ACC_ROWS, D), jnp.float32),
            pltpu.SemaphoreType.DMA,
        ],
    )
    def sc_deg(dst_hbm, ones_hbm, zeros_hbm, out_hbm, dstidx, ones_v, dacc, sem):
        c = lax.axis_index("c")
        s = lax.axis_index("s")
        wid = c * NS + s
        pltpu.sync_copy(zeros_hbm.at[pl.ds(s * ROWS_PER_SUB, ROWS_PER_SUB)],
                        dacc.at[pl.ds(s * ROWS_PER_SUB, ROWS_PER_SUB)])
        pltpu.sync_copy(ones_hbm, ones_v)
        pltpu.sync_copy(dst_hbm.at[wid], dstidx)
        plsc.subcore_barrier()

        def fire(j, carry):
            pltpu.async_copy(ones_v, dacc.at[dstidx.at[j]], sem, add=True)
            return carry

        lax.fori_loop(0, CHUNKS, fire, 0)

        def drain(j, carry):
            pltpu.make_async_copy(ones_v, dacc.at[dstidx.at[j]], sem).wait()
            return carry

        lax.fori_loop(0, CHUNKS, drain, 0)
        plsc.subcore_barrier()

        @pl.when(s < NS - 1)
        def _():
            pltpu.sync_copy(dacc.at[pl.ds(s * OUT_PER_SUB, OUT_PER_SUB)],
                            out_hbm.at[c, pl.ds(s * OUT_PER_SUB, OUT_PER_SUB)])

        @pl.when(s == NS - 1)
        def _():
            pltpu.sync_copy(dacc.at[pl.ds(15 * OUT_PER_SUB, OUT_LAST)],
                            out_hbm.at[c, pl.ds(15 * OUT_PER_SUB, OUT_LAST)])

    return sc_agg, sc_deg


def _sc_agg(*args):
    return _sc_kernels()[0](*args)


def _sc_deg(*args):
    return _sc_kernels()[1](*args)


def _dinv_from(dp0, dp1):
    deg = dp0[:, 0:1] + dp1[:, 0:1] + 1.0
    return lax.rsqrt(deg)


def _bn(x, w, b):
    mean = jnp.mean(x, axis=0, keepdims=True)
    var = jnp.mean((x - mean) ** 2, axis=0, keepdims=True)
    return (x - mean) * lax.rsqrt(var + EPS) * w + b


def _rep_store(t_ref, y):
    for k in range(REP):
        t_ref[k] = y


def _tc_feat_body(x_ref, bfw_ref, bfb_ref, Wf_ref, bf_ref,
                  bw_ref, bb_ref, W_ref, b_ref, dp0_ref, dp1_ref, t0_ref):
    dinv = _dinv_from(dp0_ref[...], dp1_ref[...])
    h = _bn(x_ref[...], bfw_ref[...], bfb_ref[...])
    h = jnp.maximum(jnp.dot(h, Wf_ref[...],
                            preferred_element_type=jnp.float32) + bf_ref[...], 0.0)
    hb = _bn(h, bw_ref[...], bb_ref[...])
    _rep_store(t0_ref, dinv * (jnp.dot(hb, W_ref[...],
                                       preferred_element_type=jnp.float32)
                               + b_ref[...]))


def _tc_mid_body(a0_ref, a1_ref, t_ref, dp0_ref, dp1_ref,
                 bw_ref, bb_ref, W_ref, b_ref, out_ref):
    dinv = _dinv_from(dp0_ref[...], dp1_ref[...])
    h = jnp.maximum(dinv * (a0_ref[...] + a1_ref[...] + t_ref[...]), 0.0)
    hb = _bn(h, bw_ref[...], bb_ref[...])
    _rep_store(out_ref, dinv * (jnp.dot(hb, W_ref[...],
                                        preferred_element_type=jnp.float32)
                                + b_ref[...]))


def _tc_final_body(a0_ref, a1_ref, t_ref, dp0_ref, dp1_ref, out_ref):
    dinv = _dinv_from(dp0_ref[...], dp1_ref[...])
    out_ref[...] = jnp.maximum(dinv * (a0_ref[...] + a1_ref[...] + t_ref[...]), 0.0)


def _tc(body, out_shape, *args):
    return pl.pallas_call(
        body, out_shape=jax.ShapeDtypeStruct(out_shape, jnp.float32))(*args)


def kernel(x, edge_index, bn_feat_w, bn_feat_b, W_feat, b_feat,
           bn_ws, bn_bs, Ws, bs):
    f32 = jnp.float32
    pad = EP - E
    src_flat = jnp.concatenate([edge_index[0], jnp.zeros((pad,), jnp.int32)])
    src_rep = src_flat + (jnp.arange(EP, dtype=jnp.int32) % REP) * N
    src3 = src_rep.reshape(NW, CHUNKS, CHUNK)
    dst3 = jnp.concatenate(
        [edge_index[1], jnp.full((pad,), N, jnp.int32)]).reshape(NW, CHUNKS, CHUNK)
    zeros_d = jnp.zeros((ACC_ROWS, D), f32)
    ones_d = jnp.ones((CHUNK, D), f32)

    degp = _sc_deg(dst3, ones_d, zeros_d)
    dp0, dp1 = degp[0], degp[1]

    row = lambda v: v.reshape(1, D)
    t = _tc(_tc_feat_body, (REP, N, D), x, row(bn_feat_w), row(bn_feat_b),
            W_feat, row(b_feat), row(bn_ws[0]), row(bn_bs[0]), Ws[0],
            row(bs[0]), dp0, dp1)
    for i in range(1, 3):
        a = _sc_agg(t.reshape(REP * N, D), src3, dst3, zeros_d)
        t = _tc(_tc_mid_body, (REP, N, D), a[0], a[1], t[0], dp0, dp1,
                row(bn_ws[i]), row(bn_bs[i]), Ws[i], row(bs[i]))
    a = _sc_agg(t.reshape(REP * N, D), src3, dst3, zeros_d)
    return _tc(_tc_final_body, (N, D), a[0], a[1], t[0], dp0, dp1)


# speedup vs baseline: 2.2925x; 1.1623x over previous
import functools

import jax
import jax.numpy as jnp
from jax import lax
from jax.experimental import pallas as pl
from jax.experimental.pallas import tpu as pltpu
from jax.experimental.pallas import tpu_sc as plsc

N = 10000
E = 320000
D = 128

NC = 2
NS = 16
NW = NC * NS
CHUNK = 128
CHUNKS = 80
EP = NW * CHUNKS * CHUNK
ACC_ROWS = 10240
ROWS_PER_SUB = ACC_ROWS // NS
OUT_PER_SUB = 624
OUT_LAST = N - 15 * OUT_PER_SUB
EPS = 1e-5
REP = 8
NBUF = 2
UNROLL = 8
HALF = CHUNKS // 2


@functools.lru_cache(maxsize=None)
def _sc_kernels():
    mesh = plsc.VectorSubcoreMesh(core_axis_name="c", subcore_axis_name="s",
                                  num_cores=NC, num_subcores=NS)

    @functools.partial(
        pl.kernel,
        out_type=jax.ShapeDtypeStruct((NC, N, D), jnp.float32),
        mesh=mesh,
        scratch_types=[
            pltpu.VMEM((HALF, CHUNK), jnp.int32),
            pltpu.VMEM((HALF, CHUNK), jnp.int32),
            pltpu.VMEM((NBUF, CHUNK, D), jnp.float32),
            pltpu.VMEM_SHARED((ACC_ROWS, D), jnp.float32),
            pltpu.SemaphoreType.DMA,
            pltpu.SemaphoreType.DMA,
        ],
    )
    def sc_agg(table_hbm, src_hbm, dst_hbm, zeros_hbm, out_hbm,
               srcidx, dstidx, rows, acc, sem0, sem1):
        sems = (sem0, sem1)
        c = lax.axis_index("c")
        s = lax.axis_index("s")
        wid = c * NS + s
        pltpu.sync_copy(zeros_hbm.at[pl.ds(s * ROWS_PER_SUB, ROWS_PER_SUB)],
                        acc.at[pl.ds(s * ROWS_PER_SUB, ROWS_PER_SUB)])
        plsc.subcore_barrier()

        for p in range(2):
            pltpu.sync_copy(src_hbm.at[wid, pl.ds(p * HALF, HALF)], srcidx)
            pltpu.sync_copy(dst_hbm.at[wid, pl.ds(p * HALF, HALF)], dstidx)
            for b in range(NBUF):
                pltpu.async_copy(table_hbm.at[srcidx.at[b]], rows.at[b], sems[b])

            def outer(o, carry):
                for i in range(UNROLL):
                    j = o * UNROLL + i
                    b = i % NBUF
                    pltpu.make_async_copy(table_hbm.at[srcidx.at[j]],
                                          rows.at[b], sems[b]).wait()
                    pltpu.sync_copy(rows.at[b], acc.at[dstidx.at[j]], add=True)

                    @pl.when(j < HALF - NBUF)
                    def _():
                        pltpu.async_copy(table_hbm.at[srcidx.at[j + NBUF]],
                                         rows.at[b], sems[b])
                return carry

            lax.fori_loop(0, HALF // UNROLL, outer, 0)
        plsc.subcore_barrier()

        @pl.when(s < NS - 1)
        def _():
            pltpu.sync_copy(acc.at[pl.ds(s * OUT_PER_SUB, OUT_PER_SUB)],
                            out_hbm.at[c, pl.ds(s * OUT_PER_SUB, OUT_PER_SUB)])

        @pl.when(s == NS - 1)
        def _():
            pltpu.sync_copy(acc.at[pl.ds(15 * OUT_PER_SUB, OUT_LAST)],
                            out_hbm.at[c, pl.ds(15 * OUT_PER_SUB, OUT_LAST)])

    @functools.partial(
        pl.kernel,
        out_type=jax.ShapeDtypeStruct((NC, N, D), jnp.float32),
        mesh=mesh,
        scratch_types=[
            pltpu.VMEM((CHUNKS, CHUNK), jnp.int32),
            pltpu.VMEM((CHUNK, D), jnp.float32),
            pltpu.VMEM_SHARED((ACC_ROWS, D), jnp.float32),
            pltpu.SemaphoreType.DMA,
        ],
    )
    def sc_deg(dst_hbm, ones_hbm, zeros_hbm, out_hbm, dstidx, ones_v, dacc, sem):
        c = lax.axis_index("c")
        s = lax.axis_index("s")
        wid = c * NS + s
        pltpu.sync_copy(zeros_hbm.at[pl.ds(s * ROWS_PER_SUB, ROWS_PER_SUB)],
                        dacc.at[pl.ds(s * ROWS_PER_SUB, ROWS_PER_SUB)])
        pltpu.sync_copy(ones_hbm, ones_v)
        pltpu.sync_copy(dst_hbm.at[wid], dstidx)
        plsc.subcore_barrier()

        def fire(j, carry):
            pltpu.async_copy(ones_v, dacc.at[dstidx.at[j]], sem, add=True)
            return carry

        lax.fori_loop(0, CHUNKS, fire, 0)

        def drain(j, carry):
            pltpu.make_async_copy(ones_v, dacc.at[dstidx.at[j]], sem).wait()
            return carry

        lax.fori_loop(0, CHUNKS, drain, 0)
        plsc.subcore_barrier()

        @pl.when(s < NS - 1)
        def _():
            pltpu.sync_copy(dacc.at[pl.ds(s * OUT_PER_SUB, OUT_PER_SUB)],
                            out_hbm.at[c, pl.ds(s * OUT_PER_SUB, OUT_PER_SUB)])

        @pl.when(s == NS - 1)
        def _():
            pltpu.sync_copy(dacc.at[pl.ds(15 * OUT_PER_SUB, OUT_LAST)],
                            out_hbm.at[c, pl.ds(15 * OUT_PER_SUB, OUT_LAST)])

    return sc_agg, sc_deg


def _sc_agg(*args):
    return _sc_kernels()[0](*args)


def _sc_deg(*args):
    return _sc_kernels()[1](*args)


def _dinv_from(dp0, dp1):
    deg = dp0[:, 0:1] + dp1[:, 0:1] + 1.0
    return lax.rsqrt(deg)


def _bn(x, w, b):
    mean = jnp.mean(x, axis=0, keepdims=True)
    var = jnp.mean((x - mean) ** 2, axis=0, keepdims=True)
    return (x - mean) * lax.rsqrt(var + EPS) * w + b


def _tc_feat_body(x_ref, bfw_ref, bfb_ref, Wf_ref, bf_ref,
                  bw_ref, bb_ref, W_ref, b_ref, dp0_ref, dp1_ref,
                  t0_ref, y_scr):
    @pl.when(pl.program_id(0) == 0)
    def _():
        dinv = _dinv_from(dp0_ref[...], dp1_ref[...])
        h = _bn(x_ref[...], bfw_ref[...], bfb_ref[...])
        h = jnp.maximum(jnp.dot(h, Wf_ref[...],
                                preferred_element_type=jnp.float32)
                        + bf_ref[...], 0.0)
        hb = _bn(h, bw_ref[...], bb_ref[...])
        y_scr[...] = dinv * (jnp.dot(hb, W_ref[...],
                                     preferred_element_type=jnp.float32)
                             + b_ref[...])

    t0_ref[0] = y_scr[...]


def _tc_mid_body(a0_ref, a1_ref, t_ref, dp0_ref, dp1_ref,
                 bw_ref, bb_ref, W_ref, b_ref, out_ref, y_scr):
    @pl.when(pl.program_id(0) == 0)
    def _():
        dinv = _dinv_from(dp0_ref[...], dp1_ref[...])
        h = jnp.maximum(dinv * (a0_ref[...] + a1_ref[...] + t_ref[...]), 0.0)
        hb = _bn(h, bw_ref[...], bb_ref[...])
        y_scr[...] = dinv * (jnp.dot(hb, W_ref[...],
                                     preferred_element_type=jnp.float32)
                             + b_ref[...])

    out_ref[0] = y_scr[...]


def _tc_final_body(a0_ref, a1_ref, t_ref, dp0_ref, dp1_ref, out_ref):
    dinv = _dinv_from(dp0_ref[...], dp1_ref[...])
    out_ref[...] = jnp.maximum(dinv * (a0_ref[...] + a1_ref[...] + t_ref[...]), 0.0)


def _tc(body, out_shape, *args):
    return pl.pallas_call(
        body, out_shape=jax.ShapeDtypeStruct(out_shape, jnp.float32))(*args)


def _tc_rep(body, *args):
    in_specs = [pl.BlockSpec(a.shape, lambda k, nd=a.ndim: (0,) * nd)
                for a in args]
    return pl.pallas_call(
        body,
        grid=(REP,),
        in_specs=in_specs,
        out_specs=pl.BlockSpec((1, N, D), lambda k: (k, 0, 0)),
        out_shape=jax.ShapeDtypeStruct((REP, N, D), jnp.float32),
        scratch_shapes=[pltpu.VMEM((N, D), jnp.float32)],
    )(*args)


def kernel(x, edge_index, bn_feat_w, bn_feat_b, W_feat, b_feat,
           bn_ws, bn_bs, Ws, bs):
    f32 = jnp.float32
    pad = EP - E
    src_flat = jnp.concatenate([edge_index[0], jnp.zeros((pad,), jnp.int32)])
    src_rep = src_flat + (jnp.arange(EP, dtype=jnp.int32) % REP) * N
    src3 = src_rep.reshape(NW, CHUNKS, CHUNK)
    dst3 = jnp.concatenate(
        [edge_index[1], jnp.full((pad,), N, jnp.int32)]).reshape(NW, CHUNKS, CHUNK)
    zeros_d = jnp.zeros((ACC_ROWS, D), f32)
    ones_d = jnp.ones((CHUNK, D), f32)

    degp = _sc_deg(dst3, ones_d, zeros_d)
    dp0, dp1 = degp[0], degp[1]

    row = lambda v: v.reshape(1, D)
    t = _tc_rep(_tc_feat_body, x, row(bn_feat_w), row(bn_feat_b),
                W_feat, row(b_feat), row(bn_ws[0]), row(bn_bs[0]), Ws[0],
                row(bs[0]), dp0, dp1)
    for i in range(1, 3):
        a = _sc_agg(t.reshape(REP * N, D), src3, dst3, zeros_d)
        t = _tc_rep(_tc_mid_body, a[0], a[1], t[0], dp0, dp1,
                    row(bn_ws[i]), row(bn_bs[i]), Ws[i], row(bs[i]))
    a = _sc_agg(t.reshape(REP * N, D), src3, dst3, zeros_d)
    return _tc(_tc_final_body, (N, D), a[0], a[1], t[0], dp0, dp1)


# speedup vs baseline: 2.5601x; 1.1167x over previous
import functools

import jax
import jax.numpy as jnp
from jax import lax
from jax.experimental import pallas as pl
from jax.experimental.pallas import tpu as pltpu
from jax.experimental.pallas import tpu_sc as plsc

N = 10000
E = 320000
D = 128

NC = 2
NS = 16
NW = NC * NS
CHUNK = 128
CHUNKS = 80
EP = NW * CHUNKS * CHUNK
ACC_ROWS = 10240
ROWS_PER_SUB = ACC_ROWS // NS
OUT_PER_SUB = 624
OUT_LAST = N - 15 * OUT_PER_SUB
EPS = 1e-5
REP = 16
NBUF = 2
UNROLL = 8
HALF = CHUNKS // 2


@functools.lru_cache(maxsize=None)
def _sc_kernels():
    mesh = plsc.VectorSubcoreMesh(core_axis_name="c", subcore_axis_name="s",
                                  num_cores=NC, num_subcores=NS)

    @functools.partial(
        pl.kernel,
        out_type=jax.ShapeDtypeStruct((NC, N, D), jnp.float32),
        mesh=mesh,
        scratch_types=[
            pltpu.VMEM((HALF, CHUNK), jnp.int32),
            pltpu.VMEM((HALF, CHUNK), jnp.int32),
            pltpu.VMEM((NBUF, CHUNK, D), jnp.float32),
            pltpu.VMEM_SHARED((ACC_ROWS, D), jnp.float32),
            pltpu.SemaphoreType.DMA,
            pltpu.SemaphoreType.DMA,
        ],
    )
    def sc_agg(table_hbm, src_hbm, dst_hbm, zeros_hbm, out_hbm,
               srcidx, dstidx, rows, acc, sem0, sem1):
        sems = (sem0, sem1)
        c = lax.axis_index("c")
        s = lax.axis_index("s")
        wid = c * NS + s
        pltpu.sync_copy(zeros_hbm.at[pl.ds(s * ROWS_PER_SUB, ROWS_PER_SUB)],
                        acc.at[pl.ds(s * ROWS_PER_SUB, ROWS_PER_SUB)])
        plsc.subcore_barrier()

        for p in range(2):
            pltpu.sync_copy(src_hbm.at[wid, pl.ds(p * HALF, HALF)], srcidx)
            pltpu.sync_copy(dst_hbm.at[wid, pl.ds(p * HALF, HALF)], dstidx)
            for b in range(NBUF):
                pltpu.async_copy(table_hbm.at[srcidx.at[b]], rows.at[b], sems[b])

            def outer(o, carry):
                for i in range(UNROLL):
                    j = o * UNROLL + i
                    b = i % NBUF
                    pltpu.make_async_copy(table_hbm.at[srcidx.at[j]],
                                          rows.at[b], sems[b]).wait()
                    pltpu.sync_copy(rows.at[b], acc.at[dstidx.at[j]], add=True)

                    @pl.when(j < HALF - NBUF)
                    def _():
                        pltpu.async_copy(table_hbm.at[srcidx.at[j + NBUF]],
                                         rows.at[b], sems[b])
                return carry

            lax.fori_loop(0, HALF // UNROLL, outer, 0)
        plsc.subcore_barrier()

        @pl.when(s < NS - 1)
        def _():
            pltpu.sync_copy(acc.at[pl.ds(s * OUT_PER_SUB, OUT_PER_SUB)],
                            out_hbm.at[c, pl.ds(s * OUT_PER_SUB, OUT_PER_SUB)])

        @pl.when(s == NS - 1)
        def _():
            pltpu.sync_copy(acc.at[pl.ds(15 * OUT_PER_SUB, OUT_LAST)],
                            out_hbm.at[c, pl.ds(15 * OUT_PER_SUB, OUT_LAST)])

    @functools.partial(
        pl.kernel,
        out_type=jax.ShapeDtypeStruct((NC, N, D), jnp.float32),
        mesh=mesh,
        scratch_types=[
            pltpu.VMEM((CHUNKS, CHUNK), jnp.int32),
            pltpu.VMEM((CHUNK, D), jnp.float32),
            pltpu.VMEM_SHARED((ACC_ROWS, D), jnp.float32),
            pltpu.SemaphoreType.DMA,
        ],
    )
    def sc_deg(dst_hbm, ones_hbm, zeros_hbm, out_hbm, dstidx, ones_v, dacc, sem):
        c = lax.axis_index("c")
        s = lax.axis_index("s")
        wid = c * NS + s
        pltpu.sync_copy(zeros_hbm.at[pl.ds(s * ROWS_PER_SUB, ROWS_PER_SUB)],
                        dacc.at[pl.ds(s * ROWS_PER_SUB, ROWS_PER_SUB)])
        pltpu.sync_copy(ones_hbm, ones_v)
        pltpu.sync_copy(dst_hbm.at[wid], dstidx)
        plsc.subcore_barrier()

        def fire(j, carry):
            pltpu.async_copy(ones_v, dacc.at[dstidx.at[j]], sem, add=True)
            return carry

        lax.fori_loop(0, CHUNKS, fire, 0)

        def drain(j, carry):
            pltpu.make_async_copy(ones_v, dacc.at[dstidx.at[j]], sem).wait()
            return carry

        lax.fori_loop(0, CHUNKS, drain, 0)
        plsc.subcore_barrier()

        @pl.when(s < NS - 1)
        def _():
            pltpu.sync_copy(dacc.at[pl.ds(s * OUT_PER_SUB, OUT_PER_SUB)],
                            out_hbm.at[c, pl.ds(s * OUT_PER_SUB, OUT_PER_SUB)])

        @pl.when(s == NS - 1)
        def _():
            pltpu.sync_copy(dacc.at[pl.ds(15 * OUT_PER_SUB, OUT_LAST)],
                            out_hbm.at[c, pl.ds(15 * OUT_PER_SUB, OUT_LAST)])

    return sc_agg, sc_deg


def _sc_agg(*args):
    return _sc_kernels()[0](*args)


def _sc_deg(*args):
    return _sc_kernels()[1](*args)


def _dinv_from(dp0, dp1):
    deg = dp0[:, 0:1] + dp1[:, 0:1] + 1.0
    return lax.rsqrt(deg)


def _bn(x, w, b):
    mean = jnp.mean(x, axis=0, keepdims=True)
    var = jnp.mean((x - mean) ** 2, axis=0, keepdims=True)
    return (x - mean) * lax.rsqrt(var + EPS) * w + b


def _tc_feat_body(x_ref, bfw_ref, bfb_ref, Wf_ref, bf_ref,
                  bw_ref, bb_ref, W_ref, b_ref, dp0_ref, dp1_ref,
                  t0_ref, y_scr):
    @pl.when(pl.program_id(0) == 0)
    def _():
        dinv = _dinv_from(dp0_ref[...], dp1_ref[...])
        h = _bn(x_ref[...], bfw_ref[...], bfb_ref[...])
        h = jnp.maximum(jnp.dot(h, Wf_ref[...],
                                preferred_element_type=jnp.float32)
                        + bf_ref[...], 0.0)
        hb = _bn(h, bw_ref[...], bb_ref[...])
        y_scr[...] = dinv * (jnp.dot(hb, W_ref[...],
                                     preferred_element_type=jnp.float32)
                             + b_ref[...])

    t0_ref[0] = y_scr[...]


def _tc_mid_body(a0_ref, a1_ref, t_ref, dp0_ref, dp1_ref,
                 bw_ref, bb_ref, W_ref, b_ref, out_ref, y_scr):
    @pl.when(pl.program_id(0) == 0)
    def _():
        dinv = _dinv_from(dp0_ref[...], dp1_ref[...])
        h = jnp.maximum(dinv * (a0_ref[...] + a1_ref[...] + t_ref[...]), 0.0)
        hb = _bn(h, bw_ref[...], bb_ref[...])
        y_scr[...] = dinv * (jnp.dot(hb, W_ref[...],
                                     preferred_element_type=jnp.float32)
                             + b_ref[...])

    out_ref[0] = y_scr[...]


def _tc_final_body(a0_ref, a1_ref, t_ref, dp0_ref, dp1_ref, out_ref):
    dinv = _dinv_from(dp0_ref[...], dp1_ref[...])
    out_ref[...] = jnp.maximum(dinv * (a0_ref[...] + a1_ref[...] + t_ref[...]), 0.0)


def _tc(body, out_shape, *args):
    return pl.pallas_call(
        body, out_shape=jax.ShapeDtypeStruct(out_shape, jnp.float32))(*args)


def _tc_rep(body, *args):
    in_specs = [pl.BlockSpec(a.shape, lambda k, nd=a.ndim: (0,) * nd)
                for a in args]
    return pl.pallas_call(
        body,
        grid=(REP,),
        in_specs=in_specs,
        out_specs=pl.BlockSpec((1, N, D), lambda k: (k, 0, 0)),
        out_shape=jax.ShapeDtypeStruct((REP, N, D), jnp.float32),
        scratch_shapes=[pltpu.VMEM((N, D), jnp.float32)],
    )(*args)


def kernel(x, edge_index, bn_feat_w, bn_feat_b, W_feat, b_feat,
           bn_ws, bn_bs, Ws, bs):
    f32 = jnp.float32
    pad = EP - E
    src_flat = jnp.concatenate([edge_index[0], jnp.zeros((pad,), jnp.int32)])
    src_rep = src_flat + (jnp.arange(EP, dtype=jnp.int32) % REP) * N
    src3 = src_rep.reshape(NW, CHUNKS, CHUNK)
    dst3 = jnp.concatenate(
        [edge_index[1], jnp.full((pad,), N, jnp.int32)]).reshape(NW, CHUNKS, CHUNK)
    zeros_d = jnp.zeros((ACC_ROWS, D), f32)
    ones_d = jnp.ones((CHUNK, D), f32)

    degp = _sc_deg(dst3, ones_d, zeros_d)
    dp0, dp1 = degp[0], degp[1]

    row = lambda v: v.reshape(1, D)
    t = _tc_rep(_tc_feat_body, x, row(bn_feat_w), row(bn_feat_b),
                W_feat, row(b_feat), row(bn_ws[0]), row(bn_bs[0]), Ws[0],
                row(bs[0]), dp0, dp1)
    for i in range(1, 3):
        a = _sc_agg(t.reshape(REP * N, D), src3, dst3, zeros_d)
        t = _tc_rep(_tc_mid_body, a[0], a[1], t[0], dp0, dp1,
                    row(bn_ws[i]), row(bn_bs[i]), Ws[i], row(bs[i]))
    a = _sc_agg(t.reshape(REP * N, D), src3, dst3, zeros_d)
    return _tc(_tc_final_body, (N, D), a[0], a[1], t[0], dp0, dp1)
